# baseline (speedup 1.0000x reference)
import jax
import jax.numpy as jnp
from jax.experimental import pallas as pl
from jax.experimental.pallas import tpu as pltpu

_TILE_ROWS = 16


def _zero_fill_body(out_ref, tile_ref, sems):
    n = out_ref.shape[0] // _TILE_ROWS
    tile_ref[...] = jnp.zeros(tile_ref.shape, tile_ref.dtype)
    copies = [
        pltpu.make_async_copy(
            tile_ref,
            out_ref.at[pl.ds(i * _TILE_ROWS, _TILE_ROWS), :],
            sems.at[i],
        )
        for i in range(n)
    ]
    for c in copies:
        c.start()
    for c in copies:
        c.wait()


def kernel(x, idx):
    b, frame_len = x.shape
    del idx
    n = b // _TILE_ROWS
    return pl.pallas_call(
        _zero_fill_body,
        out_specs=pl.BlockSpec(memory_space=pl.ANY),
        out_shape=jax.ShapeDtypeStruct((b, frame_len), x.dtype),
        scratch_shapes=[
            pltpu.VMEM((_TILE_ROWS, frame_len), x.dtype),
            pltpu.SemaphoreType.DMA((n,)),
        ],
    )()


# speedup vs baseline: 1.0281x; 1.0281x over previous
import jax
import jax.numpy as jnp
from jax.experimental import pallas as pl
from jax.experimental.pallas import tpu as pltpu

_TILE_ROWS = 32


def _zero_fill_body(out_ref, tile_ref, sems):
    n = out_ref.shape[0] // _TILE_ROWS
    tile_ref[...] = jnp.zeros(tile_ref.shape, tile_ref.dtype)
    copies = [
        pltpu.make_async_copy(
            tile_ref,
            out_ref.at[pl.ds(i * _TILE_ROWS, _TILE_ROWS), :],
            sems.at[i],
        )
        for i in range(n)
    ]
    for c in copies:
        c.start()
    for c in copies:
        c.wait()


def kernel(x, idx):
    b, frame_len = x.shape
    del idx
    n = b // _TILE_ROWS
    return pl.pallas_call(
        _zero_fill_body,
        out_specs=pl.BlockSpec(memory_space=pl.ANY),
        out_shape=jax.ShapeDtypeStruct((b, frame_len), x.dtype),
        scratch_shapes=[
            pltpu.VMEM((_TILE_ROWS, frame_len), x.dtype),
            pltpu.SemaphoreType.DMA((n,)),
        ],
    )()
